# baseline (speedup 1.0000x reference)
import functools
import jax
import jax.numpy as jnp
from jax import lax
from jax.experimental import pallas as pl
from jax.experimental.pallas import tpu as pltpu
from jax.experimental.pallas import tpu_sc as plsc

N_PAPER = 10000
N_LABEL = 1000
D_FEAT = 128
HIDDEN = 128
EMB = 64
E_CITES = 320000
E_IS = 10000

NC, NS = 2, 16
NW = NC * NS
OCH = 64


def _pad_rows(n):
    return (NS * 8) * (-(-(n + 1) // (NS * 8)))


def _ceil_to(x, m):
    return m * (-(-x // m))


CHUNK = 128


def _job_pad(n_edges):
    per_w = _ceil_to(-(-n_edges // NW), 2 * CHUNK)
    return per_w, per_w * NW


def _make_sc_multi(jobs, deg_mode=False):
    d = 16 if deg_mode else 128
    meta = []
    for (n_dst, n_edges) in jobs:
        per_w, _ = _job_pad(n_edges)
        meta.append((_pad_rows(n_dst), per_w))

    mesh = plsc.VectorSubcoreMesh(core_axis_name="c", subcore_axis_name="s",
                                  num_cores=NC, num_subcores=NS)
    scratch = (
        [
            pltpu.VMEM((CHUNK,), jnp.int32),
            pltpu.VMEM((CHUNK,), jnp.int32),
            pltpu.VMEM((CHUNK,), jnp.int32),
            pltpu.VMEM((CHUNK, d), jnp.float32),
            pltpu.VMEM((CHUNK, d), jnp.float32),
            pltpu.VMEM((OCH, d), jnp.float32),
            pltpu.SemaphoreType.DMA,
        ]
        + [pltpu.VMEM_SHARED((np_dst, d), jnp.float32) for np_dst, _ in meta]
    )

    nj = len(jobs)

    def body(*refs):
        if deg_mode:
            dst1d, out0 = refs[0], refs[1]
            sc = refs[2:]
            outs = (out0,)
            tabs = (None,)
            srcs = (None,)
            dsts = (dst1d,)
        else:
            tabs = refs[0:3 * nj:3]
            srcs = refs[1:3 * nj:3]
            dsts = refs[2:3 * nj:3]
            outs = refs[3 * nj:4 * nj]
            sc = refs[4 * nj:]
        (idx_s, idx_d0, idx_d1, rows0, rows1, obuf, sem_s) = sc[0:7]
        idx_d = (idx_d0, idx_d1)
        rowsb = (rows0, rows1)
        rows = rows0
        accs = sc[7:]
        c = lax.axis_index("c")
        s = lax.axis_index("s")
        wid = c * NS + s

        zeros16 = jnp.zeros((16,), jnp.float32)

        def zrow(i, carry):
            for j in range(d // 16):
                obuf[i, pl.ds(j * 16, 16)] = zeros16
            return carry
        lax.fori_loop(0, OCH, zrow, 0)

        if deg_mode:
            ones16 = jnp.ones((16,), jnp.float32)

            def orow(i, carry):
                for j in range(d // 16):
                    rows[i, pl.ds(j * 16, 16)] = ones16
                return carry
            lax.fori_loop(0, CHUNK, orow, 0)

        for (np_dst, _), acc in zip(meta, accs):
            rows_sub = np_dst // NS
            r0 = s * rows_sub
            nfull = rows_sub // OCH
            rem = rows_sub % OCH

            def zcopy(i, carry, acc=acc, r0=r0):
                pltpu.sync_copy(obuf, acc.at[pl.ds(r0 + i * OCH, OCH)])
                return carry
            lax.fori_loop(0, nfull, zcopy, 0)
            if rem:
                pltpu.sync_copy(obuf.at[pl.ds(0, rem)],
                                acc.at[pl.ds(r0 + nfull * OCH, rem)])
        plsc.subcore_barrier()

        for ji in range(nj):
            np_dst, per_w = meta[ji]
            table, src3d, dst3d, acc = tabs[ji], srcs[ji], dsts[ji], accs[ji]
            nchunk = per_w // CHUNK
            base = wid * per_w

            def scat_desc(ph, acc=acc):
                rb = rows0 if (deg_mode or ph == 0) else rows1
                return pltpu.make_async_copy(rb, acc.at[idx_d[ph]], sem_s)

            def pair_body(t, carry, acc=acc, table=table,
                          src3d=src3d, dst3d=dst3d, scat_desc=scat_desc):
                for ph in range(2):
                    k = 2 * t + ph

                    @pl.when(t > 0)
                    def _(ph=ph):
                        scat_desc(ph).wait()
                    b = base + k * CHUNK
                    pltpu.sync_copy(dst3d.at[pl.ds(b, CHUNK)], idx_d[ph])
                    if not deg_mode:
                        pltpu.sync_copy(src3d.at[pl.ds(b, CHUNK)], idx_s)
                        pltpu.sync_copy(table.at[idx_s], rowsb[ph])
                    scat_desc(ph).start(add=True)
                return carry
            lax.fori_loop(0, nchunk // 2, pair_body, 0)
            scat_desc(0).wait()
            scat_desc(1).wait()

        plsc.subcore_barrier()

        for (np_dst, _), acc, out in zip(meta, accs, outs):
            rows_sub = np_dst // NS
            r0 = s * rows_sub
            nfull = rows_sub // OCH
            rem = rows_sub % OCH

            def ocopy(i, carry, acc=acc, out=out, r0=r0):
                pltpu.sync_copy(acc.at[pl.ds(r0 + i * OCH, OCH)], obuf)
                pltpu.sync_copy(obuf,
                                out.at[c, pl.ds(r0 + i * OCH, OCH)])
                return carry
            lax.fori_loop(0, nfull, ocopy, 0)
            if rem:
                pltpu.sync_copy(acc.at[pl.ds(r0 + nfull * OCH, rem)],
                                obuf.at[pl.ds(0, rem)])
                pltpu.sync_copy(obuf.at[pl.ds(0, rem)],
                                out.at[c, pl.ds(r0 + nfull * OCH, rem)])

    out_type = tuple(jax.ShapeDtypeStruct((NC, np_dst, d), jnp.float32)
                     for np_dst, _ in meta)
    if len(out_type) == 1:
        out_type = out_type[0]
    fn = pl.kernel(body, out_type=out_type, mesh=mesh, scratch_types=scratch)
    return fn


def _pad_edges(src, dst, n_edges, dummy_row):
    _, e_pad = _job_pad(n_edges)
    pe = e_pad - src.shape[0]
    src = jnp.concatenate([src, jnp.zeros((pe,), jnp.int32)])
    dst = jnp.concatenate([dst, jnp.full((pe,), dummy_row, jnp.int32)])
    return src, dst


_BP = 1000


def _dinv_from(degp_ref):
    deg = degp_ref[0, :, 0:1] + degp_ref[1, :, 0:1] + 1.0
    return lax.rsqrt(deg)


def _pre_paper_body(x_ref, degp_ref, wcat_ref, wg_ref, his_ref, rr_ref, hs_ref):
    x = x_ref[...]
    dinv = _dinv_from(degp_ref)
    y = jnp.dot(x, wcat_ref[...], preferred_element_type=jnp.float32)
    his_ref[...] = y[:, :HIDDEN]
    rr_ref[...] = y[:, HIDDEN:]
    hs_ref[...] = jnp.dot(x * dinv, wg_ref[...],
                          preferred_element_type=jnp.float32)


def _pre_label_body(x_ref, wcat_ref, hrev_ref, ri_ref):
    y = jnp.dot(x_ref[...], wcat_ref[...], preferred_element_type=jnp.float32)
    hrev_ref[...] = y[:, :HIDDEN]
    ri_ref[...] = y[:, HIDDEN:]


def _mid_paper_body(sc_ref, sr_ref, hs_ref, rr_ref, degp_ref, bg_ref, brr_ref,
                    zp1_ref, zp1s_ref):
    dinv = _dinv_from(degp_ref)
    gcn = dinv * (sc_ref[0] + sc_ref[1] + hs_ref[...]) + bg_ref[...]
    rev = sr_ref[0] + sr_ref[1] + brr_ref[...] + rr_ref[...]
    zp1 = jax.nn.relu(0.5 * (gcn + rev))
    zp1_ref[...] = zp1
    zp1s_ref[...] = zp1 * dinv


def _mid_label_body(sis_ref, ri_ref, bri_ref, zl1_ref):
    zl1_ref[...] = jax.nn.relu(sis_ref[0] + sis_ref[1] + bri_ref[...]
                               + ri_ref[...])


def _post_paper_body(sc_ref, sr_ref, zp1s_ref, zp1_ref, degp_ref, bg_ref,
                     brr_ref, wg2_ref, wrr2_ref, wtr2_ref, zp_ref):
    dinv = _dinv_from(degp_ref)
    a = sc_ref[0] + sc_ref[1] + zp1s_ref[...]
    gcn = dinv * jnp.dot(a, wg2_ref[...], preferred_element_type=jnp.float32) \
        + bg_ref[...]
    rev = jnp.dot(sr_ref[0] + sr_ref[1], wrr2_ref[...],
                  preferred_element_type=jnp.float32) + brr_ref[...] \
        + jnp.dot(zp1_ref[...], wtr2_ref[...],
                  preferred_element_type=jnp.float32)
    zp_ref[...] = 0.5 * (gcn + rev)


def _post_label_body(sis_ref, zl1_ref, bri_ref, wri2_ref, wti2_ref, zl_ref):
    zl_ref[...] = jnp.dot(sis_ref[0] + sis_ref[1], wri2_ref[...],
                          preferred_element_type=jnp.float32) \
        + bri_ref[...] \
        + jnp.dot(zl1_ref[...], wti2_ref[...],
                  preferred_element_type=jnp.float32)


def _row_spec(d):
    return pl.BlockSpec((_BP, d), lambda i: (i, 0))


def _part_spec(d):
    return pl.BlockSpec((NC, _BP, d), lambda i: (0, i, 0))


def _full_spec(shape):
    nz = len(shape)
    return pl.BlockSpec(shape, lambda *a: (0,) * nz)


def kernel(x_paper, x_label, edge_index_cites, is_src, is_dst, rev_src, rev_dst,
           W_gcn1, b_gcn1, Wrel_is1, brel_is1, Wroot_is1, Wrel_rev1, brel_rev1,
           Wroot_rev1, W_gcn2, b_gcn2, Wrel_is2, brel_is2, Wroot_is2,
           Wrel_rev2, brel_rev2, Wroot_rev2):
    f32 = jnp.float32
    cit_src = edge_index_cites[0]
    cit_dst = edge_index_cites[1]

    deg_fn = _make_sc_multi([(N_PAPER, E_CITES)], deg_mode=True)
    cites_fn = _make_sc_multi([(N_PAPER, E_CITES)])
    rev_fn = _make_sc_multi([(N_PAPER, E_IS)])
    is_fn = _make_sc_multi([(N_LABEL, E_IS)])

    csrc_p, cdst_p = _pad_edges(cit_src, cit_dst, E_CITES, N_PAPER)
    rsrc_p, rdst_p = _pad_edges(rev_src, rev_dst, E_IS, N_PAPER)
    isrc_p, idst_p = _pad_edges(is_src, is_dst, E_IS, N_LABEL)

    degp_full = deg_fn(cdst_p)
    degp = degp_full[:, :N_PAPER, :]

    wcat_p1 = jnp.concatenate([Wrel_is1, Wroot_rev1], axis=1)
    grid_p = (N_PAPER // _BP,)
    his1, rr1, hs1 = pl.pallas_call(
        _pre_paper_body,
        grid=grid_p,
        in_specs=[_row_spec(D_FEAT),
                  pl.BlockSpec((NC, _BP, 16), lambda i: (0, i, 0)),
                  _full_spec((D_FEAT, 2 * HIDDEN)),
                  _full_spec((D_FEAT, HIDDEN))],
        out_specs=[_row_spec(HIDDEN)] * 3,
        out_shape=[jax.ShapeDtypeStruct((N_PAPER, HIDDEN), f32)] * 3,
    )(x_paper, degp, wcat_p1, W_gcn1)

    wcat_l1 = jnp.concatenate([Wrel_rev1, Wroot_is1], axis=1)
    hrev1, ri1 = pl.pallas_call(
        _pre_label_body,
        in_specs=[_full_spec((N_LABEL, D_FEAT)),
                  _full_spec((D_FEAT, 2 * HIDDEN))],
        out_specs=[_full_spec((N_LABEL, HIDDEN))] * 2,
        out_shape=[jax.ShapeDtypeStruct((N_LABEL, HIDDEN), f32)] * 2,
    )(x_label, wcat_l1)

    sc1 = cites_fn(hs1, csrc_p, cdst_p)[:, :N_PAPER, :]
    sr1 = rev_fn(hrev1, rsrc_p, rdst_p)[:, :N_PAPER, :]
    sis1 = is_fn(his1, isrc_p, idst_p)[:, :N_LABEL, :]

    zp1, zp1s = pl.pallas_call(
        _mid_paper_body,
        grid=grid_p,
        in_specs=[_part_spec(HIDDEN), _part_spec(HIDDEN),
                  _row_spec(HIDDEN), _row_spec(HIDDEN),
                  pl.BlockSpec((NC, _BP, 16), lambda i: (0, i, 0)),
                  _full_spec((1, HIDDEN)), _full_spec((1, HIDDEN))],
        out_specs=[_row_spec(HIDDEN)] * 2,
        out_shape=[jax.ShapeDtypeStruct((N_PAPER, HIDDEN), f32)] * 2,
    )(sc1, sr1, hs1, rr1, degp, b_gcn1.reshape(1, -1),
      brel_rev1.reshape(1, -1))

    zl1 = pl.pallas_call(
        _mid_label_body,
        in_specs=[_full_spec((NC, N_LABEL, HIDDEN)),
                  _full_spec((N_LABEL, HIDDEN)),
                  _full_spec((1, HIDDEN))],
        out_specs=_full_spec((N_LABEL, HIDDEN)),
        out_shape=jax.ShapeDtypeStruct((N_LABEL, HIDDEN), f32),
    )(sis1, ri1, brel_is1.reshape(1, -1))

    sc2 = cites_fn(zp1s, csrc_p, cdst_p)[:, :N_PAPER, :]
    sr2 = rev_fn(zl1, rsrc_p, rdst_p)[:, :N_PAPER, :]
    sis2 = is_fn(zp1, isrc_p, idst_p)[:, :N_LABEL, :]

    zp2 = pl.pallas_call(
        _post_paper_body,
        grid=grid_p,
        in_specs=[_part_spec(HIDDEN), _part_spec(HIDDEN),
                  _row_spec(HIDDEN), _row_spec(HIDDEN),
                  pl.BlockSpec((NC, _BP, 16), lambda i: (0, i, 0)),
                  _full_spec((1, EMB)), _full_spec((1, EMB)),
                  _full_spec((HIDDEN, EMB)), _full_spec((HIDDEN, EMB)),
                  _full_spec((HIDDEN, EMB))],
        out_specs=_row_spec(EMB),
        out_shape=jax.ShapeDtypeStruct((N_PAPER, EMB), f32),
    )(sc2, sr2, zp1s, zp1, degp, b_gcn2.reshape(1, -1),
      brel_rev2.reshape(1, -1), W_gcn2, Wrel_rev2, Wroot_rev2)

    zl2 = pl.pallas_call(
        _post_label_body,
        in_specs=[_full_spec((NC, N_LABEL, HIDDEN)),
                  _full_spec((N_LABEL, HIDDEN)),
                  _full_spec((1, EMB)),
                  _full_spec((HIDDEN, EMB)), _full_spec((HIDDEN, EMB))],
        out_specs=_full_spec((N_LABEL, EMB)),
        out_shape=jax.ShapeDtypeStruct((N_LABEL, EMB), f32),
    )(sis2, zl1, brel_is2.reshape(1, -1), Wrel_is2, Wroot_is2)

    return zp2, zl2


# speedup vs baseline: 1.6187x; 1.6187x over previous
import functools
import jax
import jax.numpy as jnp
from jax import lax
from jax.experimental import pallas as pl
from jax.experimental.pallas import tpu as pltpu
from jax.experimental.pallas import tpu_sc as plsc

N_PAPER = 10000
N_LABEL = 1000
D_FEAT = 128
HIDDEN = 128
EMB = 64
E_CITES = 320000
E_IS = 10000

NC, NS = 2, 16
NW = NC * NS
OCH = 128


def _pad_rows(n):
    return (NS * 8) * (-(-(n + 1) // (NS * 8)))


def _ceil_to(x, m):
    return m * (-(-x // m))


CHUNK = 128


def _job_pad(n_edges):
    per_w = _ceil_to(-(-n_edges // NW), CHUNK)
    return per_w, per_w * NW


def _make_sc_multi(jobs, deg_mode=False):
    d = 16 if deg_mode else 128
    meta = []
    for (n_dst, n_edges) in jobs:
        per_w, _ = _job_pad(n_edges)
        meta.append((_pad_rows(n_dst), per_w))

    mesh = plsc.VectorSubcoreMesh(core_axis_name="c", subcore_axis_name="s",
                                  num_cores=NC, num_subcores=NS)
    scratch = (
        [
            pltpu.VMEM((CHUNK,), jnp.int32),
            pltpu.VMEM((CHUNK,), jnp.int32),
            pltpu.VMEM((CHUNK, d), jnp.float32),
            pltpu.VMEM((OCH, d), jnp.float32),
        ]
        + [pltpu.VMEM_SHARED((np_dst, d), jnp.float32) for np_dst, _ in meta]
    )

    nj = len(jobs)

    def body(*refs):
        if deg_mode:
            dst1d, out0 = refs[0], refs[1]
            sc = refs[2:]
            outs = (out0,)
            tabs = (None,)
            srcs = (None,)
            dsts = (dst1d,)
        else:
            tabs = refs[0:3 * nj:3]
            srcs = refs[1:3 * nj:3]
            dsts = refs[2:3 * nj:3]
            outs = refs[3 * nj:4 * nj]
            sc = refs[4 * nj:]
        (idx_s, idx_d, rows, obuf) = sc[0:4]
        accs = sc[4:]
        c = lax.axis_index("c")
        s = lax.axis_index("s")
        wid = c * NS + s

        zeros16 = jnp.zeros((16,), jnp.float32)

        def zrow(i, carry):
            for j in range(d // 16):
                obuf[i, pl.ds(j * 16, 16)] = zeros16
            return carry
        lax.fori_loop(0, OCH, zrow, 0)

        if deg_mode:
            ones16 = jnp.ones((16,), jnp.float32)

            def orow(i, carry):
                for j in range(d // 16):
                    rows[i, pl.ds(j * 16, 16)] = ones16
                return carry
            lax.fori_loop(0, CHUNK, orow, 0)

        for (np_dst, _), acc in zip(meta, accs):
            rows_sub = np_dst // NS
            r0 = s * rows_sub
            nfull = rows_sub // OCH
            rem = rows_sub % OCH

            def zcopy(i, carry, acc=acc, r0=r0):
                pltpu.sync_copy(obuf, acc.at[pl.ds(r0 + i * OCH, OCH)])
                return carry
            lax.fori_loop(0, nfull, zcopy, 0)
            if rem:
                pltpu.sync_copy(obuf.at[pl.ds(0, rem)],
                                acc.at[pl.ds(r0 + nfull * OCH, rem)])
        plsc.subcore_barrier()

        for ji in range(nj):
            np_dst, per_w = meta[ji]
            table, src3d, dst3d, acc = tabs[ji], srcs[ji], dsts[ji], accs[ji]
            nchunk = per_w // CHUNK
            base = wid * per_w

            def chunk_body(k, carry, acc=acc, table=table,
                           src3d=src3d, dst3d=dst3d):
                b = base + k * CHUNK
                pltpu.sync_copy(dst3d.at[pl.ds(b, CHUNK)], idx_d)
                if not deg_mode:
                    pltpu.sync_copy(src3d.at[pl.ds(b, CHUNK)], idx_s)
                    pltpu.sync_copy(table.at[idx_s], rows)
                pltpu.sync_copy(rows, acc.at[idx_d], add=True)
                return carry
            lax.fori_loop(0, nchunk, chunk_body, 0)

        plsc.subcore_barrier()

        for (np_dst, _), acc, out in zip(meta, accs, outs):
            rows_sub = np_dst // NS
            r0 = s * rows_sub
            nfull = rows_sub // OCH
            rem = rows_sub % OCH

            def ocopy(i, carry, acc=acc, out=out, r0=r0):
                pltpu.sync_copy(acc.at[pl.ds(r0 + i * OCH, OCH)], obuf)
                pltpu.sync_copy(obuf,
                                out.at[c, pl.ds(r0 + i * OCH, OCH)])
                return carry
            lax.fori_loop(0, nfull, ocopy, 0)
            if rem:
                pltpu.sync_copy(acc.at[pl.ds(r0 + nfull * OCH, rem)],
                                obuf.at[pl.ds(0, rem)])
                pltpu.sync_copy(obuf.at[pl.ds(0, rem)],
                                out.at[c, pl.ds(r0 + nfull * OCH, rem)])

    out_type = tuple(jax.ShapeDtypeStruct((NC, np_dst, d), jnp.float32)
                     for np_dst, _ in meta)
    if len(out_type) == 1:
        out_type = out_type[0]
    fn = pl.kernel(body, out_type=out_type, mesh=mesh, scratch_types=scratch)
    return fn


def _pad_edges(src, dst, n_edges, dummy_row):
    _, e_pad = _job_pad(n_edges)
    pe = e_pad - src.shape[0]
    src = jnp.concatenate([src, jnp.zeros((pe,), jnp.int32)])
    dst = jnp.concatenate([dst, jnp.full((pe,), dummy_row, jnp.int32)])
    return src, dst


_BP = 1000


def _dinv_from(degp_ref):
    deg = degp_ref[0, :, 0:1] + degp_ref[1, :, 0:1] + 1.0
    return lax.rsqrt(deg)


def _pre_paper_body(x_ref, degp_ref, wcat_ref, wg_ref, his_ref, rr_ref, hs_ref):
    x = x_ref[...]
    dinv = _dinv_from(degp_ref)
    y = jnp.dot(x, wcat_ref[...], preferred_element_type=jnp.float32)
    his_ref[...] = y[:, :HIDDEN]
    rr_ref[...] = y[:, HIDDEN:]
    hs_ref[...] = jnp.dot(x * dinv, wg_ref[...],
                          preferred_element_type=jnp.float32)


def _pre_label_body(x_ref, wcat_ref, hrev_ref, ri_ref):
    y = jnp.dot(x_ref[...], wcat_ref[...], preferred_element_type=jnp.float32)
    hrev_ref[...] = y[:, :HIDDEN]
    ri_ref[...] = y[:, HIDDEN:]


def _mid_paper_body(sc_ref, sr_ref, hs_ref, rr_ref, degp_ref, bg_ref, brr_ref,
                    zp1_ref, zp1s_ref):
    dinv = _dinv_from(degp_ref)
    gcn = dinv * (sc_ref[0] + sc_ref[1] + hs_ref[...]) + bg_ref[...]
    rev = sr_ref[0] + sr_ref[1] + brr_ref[...] + rr_ref[...]
    zp1 = jax.nn.relu(0.5 * (gcn + rev))
    zp1_ref[...] = zp1
    zp1s_ref[...] = zp1 * dinv


def _mid_label_body(sis_ref, ri_ref, bri_ref, zl1_ref):
    zl1_ref[...] = jax.nn.relu(sis_ref[0] + sis_ref[1] + bri_ref[...]
                               + ri_ref[...])


def _post_paper_body(sc_ref, sr_ref, zp1s_ref, zp1_ref, degp_ref, bg_ref,
                     brr_ref, wg2_ref, wrr2_ref, wtr2_ref, zp_ref):
    dinv = _dinv_from(degp_ref)
    a = sc_ref[0] + sc_ref[1] + zp1s_ref[...]
    gcn = dinv * jnp.dot(a, wg2_ref[...], preferred_element_type=jnp.float32) \
        + bg_ref[...]
    rev = jnp.dot(sr_ref[0] + sr_ref[1], wrr2_ref[...],
                  preferred_element_type=jnp.float32) + brr_ref[...] \
        + jnp.dot(zp1_ref[...], wtr2_ref[...],
                  preferred_element_type=jnp.float32)
    zp_ref[...] = 0.5 * (gcn + rev)


def _post_label_body(sis_ref, zl1_ref, bri_ref, wri2_ref, wti2_ref, zl_ref):
    zl_ref[...] = jnp.dot(sis_ref[0] + sis_ref[1], wri2_ref[...],
                          preferred_element_type=jnp.float32) \
        + bri_ref[...] \
        + jnp.dot(zl1_ref[...], wti2_ref[...],
                  preferred_element_type=jnp.float32)


def _row_spec(d):
    return pl.BlockSpec((_BP, d), lambda i: (i, 0))


def _part_spec(d):
    return pl.BlockSpec((NC, _BP, d), lambda i: (0, i, 0))


def _full_spec(shape):
    nz = len(shape)
    return pl.BlockSpec(shape, lambda *a: (0,) * nz)


def kernel(x_paper, x_label, edge_index_cites, is_src, is_dst, rev_src, rev_dst,
           W_gcn1, b_gcn1, Wrel_is1, brel_is1, Wroot_is1, Wrel_rev1, brel_rev1,
           Wroot_rev1, W_gcn2, b_gcn2, Wrel_is2, brel_is2, Wroot_is2,
           Wrel_rev2, brel_rev2, Wroot_rev2):
    f32 = jnp.float32
    cit_src = edge_index_cites[0]
    cit_dst = edge_index_cites[1]

    deg_fn = _make_sc_multi([(N_PAPER, E_CITES)], deg_mode=True)
    cites_fn = _make_sc_multi([(N_PAPER, E_CITES)])
    rev_fn = _make_sc_multi([(N_PAPER, E_IS)])
    is_fn = _make_sc_multi([(N_LABEL, E_IS)])

    csrc_p, cdst_p = _pad_edges(cit_src, cit_dst, E_CITES, N_PAPER)
    rsrc_p, rdst_p = _pad_edges(rev_src, rev_dst, E_IS, N_PAPER)
    isrc_p, idst_p = _pad_edges(is_src, is_dst, E_IS, N_LABEL)

    degp_full = deg_fn(cdst_p)
    degp = degp_full[:, :N_PAPER, :]

    wcat_p1 = jnp.concatenate([Wrel_is1, Wroot_rev1], axis=1)
    grid_p = (N_PAPER // _BP,)
    his1, rr1, hs1 = pl.pallas_call(
        _pre_paper_body,
        grid=grid_p,
        in_specs=[_row_spec(D_FEAT),
                  pl.BlockSpec((NC, _BP, 16), lambda i: (0, i, 0)),
                  _full_spec((D_FEAT, 2 * HIDDEN)),
                  _full_spec((D_FEAT, HIDDEN))],
        out_specs=[_row_spec(HIDDEN)] * 3,
        out_shape=[jax.ShapeDtypeStruct((N_PAPER, HIDDEN), f32)] * 3,
    )(x_paper, degp, wcat_p1, W_gcn1)

    wcat_l1 = jnp.concatenate([Wrel_rev1, Wroot_is1], axis=1)
    hrev1, ri1 = pl.pallas_call(
        _pre_label_body,
        in_specs=[_full_spec((N_LABEL, D_FEAT)),
                  _full_spec((D_FEAT, 2 * HIDDEN))],
        out_specs=[_full_spec((N_LABEL, HIDDEN))] * 2,
        out_shape=[jax.ShapeDtypeStruct((N_LABEL, HIDDEN), f32)] * 2,
    )(x_label, wcat_l1)

    sc1 = cites_fn(hs1, csrc_p, cdst_p)[:, :N_PAPER, :]
    sr1 = rev_fn(hrev1, rsrc_p, rdst_p)[:, :N_PAPER, :]
    sis1 = is_fn(his1, isrc_p, idst_p)[:, :N_LABEL, :]

    zp1, zp1s = pl.pallas_call(
        _mid_paper_body,
        grid=grid_p,
        in_specs=[_part_spec(HIDDEN), _part_spec(HIDDEN),
                  _row_spec(HIDDEN), _row_spec(HIDDEN),
                  pl.BlockSpec((NC, _BP, 16), lambda i: (0, i, 0)),
                  _full_spec((1, HIDDEN)), _full_spec((1, HIDDEN))],
        out_specs=[_row_spec(HIDDEN)] * 2,
        out_shape=[jax.ShapeDtypeStruct((N_PAPER, HIDDEN), f32)] * 2,
    )(sc1, sr1, hs1, rr1, degp, b_gcn1.reshape(1, -1),
      brel_rev1.reshape(1, -1))

    zl1 = pl.pallas_call(
        _mid_label_body,
        in_specs=[_full_spec((NC, N_LABEL, HIDDEN)),
                  _full_spec((N_LABEL, HIDDEN)),
                  _full_spec((1, HIDDEN))],
        out_specs=_full_spec((N_LABEL, HIDDEN)),
        out_shape=jax.ShapeDtypeStruct((N_LABEL, HIDDEN), f32),
    )(sis1, ri1, brel_is1.reshape(1, -1))

    sc2 = cites_fn(zp1s, csrc_p, cdst_p)[:, :N_PAPER, :]
    sr2 = rev_fn(zl1, rsrc_p, rdst_p)[:, :N_PAPER, :]
    sis2 = is_fn(zp1, isrc_p, idst_p)[:, :N_LABEL, :]

    zp2 = pl.pallas_call(
        _post_paper_body,
        grid=grid_p,
        in_specs=[_part_spec(HIDDEN), _part_spec(HIDDEN),
                  _row_spec(HIDDEN), _row_spec(HIDDEN),
                  pl.BlockSpec((NC, _BP, 16), lambda i: (0, i, 0)),
                  _full_spec((1, EMB)), _full_spec((1, EMB)),
                  _full_spec((HIDDEN, EMB)), _full_spec((HIDDEN, EMB)),
                  _full_spec((HIDDEN, EMB))],
        out_specs=_row_spec(EMB),
        out_shape=jax.ShapeDtypeStruct((N_PAPER, EMB), f32),
    )(sc2, sr2, zp1s, zp1, degp, b_gcn2.reshape(1, -1),
      brel_rev2.reshape(1, -1), W_gcn2, Wrel_rev2, Wroot_rev2)

    zl2 = pl.pallas_call(
        _post_label_body,
        in_specs=[_full_spec((NC, N_LABEL, HIDDEN)),
                  _full_spec((N_LABEL, HIDDEN)),
                  _full_spec((1, EMB)),
                  _full_spec((HIDDEN, EMB)), _full_spec((HIDDEN, EMB))],
        out_specs=_full_spec((N_LABEL, EMB)),
        out_shape=jax.ShapeDtypeStruct((N_LABEL, EMB), f32),
    )(sis2, zl1, brel_is2.reshape(1, -1), Wrel_is2, Wroot_is2)

    return zp2, zl2


# speedup vs baseline: 1.6354x; 1.0103x over previous
import functools
import jax
import jax.numpy as jnp
from jax import lax
from jax.experimental import pallas as pl
from jax.experimental.pallas import tpu as pltpu
from jax.experimental.pallas import tpu_sc as plsc

N_PAPER = 10000
N_LABEL = 1000
D_FEAT = 128
HIDDEN = 128
EMB = 64
E_CITES = 320000
E_IS = 10000

NC, NS = 2, 16
NW = NC * NS
OCH = 128


def _pad_rows(n):
    return (NS * 8) * (-(-(n + 1) // (NS * 8)))


def _ceil_to(x, m):
    return m * (-(-x // m))


CHUNK = 128


def _job_pad(n_edges):
    per_w = _ceil_to(-(-n_edges // NW), CHUNK)
    return per_w, per_w * NW


def _make_sc_multi(jobs, deg_mode=False):
    d = 16 if deg_mode else 128
    meta = []
    for (n_dst, n_edges) in jobs:
        per_w, _ = _job_pad(n_edges)
        meta.append((_pad_rows(n_dst), per_w))

    mesh = plsc.VectorSubcoreMesh(core_axis_name="c", subcore_axis_name="s",
                                  num_cores=NC, num_subcores=NS)
    scratch = (
        [
            pltpu.VMEM((CHUNK,), jnp.int32),
            pltpu.VMEM((CHUNK,), jnp.int32),
            pltpu.VMEM((CHUNK, d), jnp.float32),
            pltpu.VMEM((OCH, d), jnp.float32),
        ]
        + [pltpu.VMEM_SHARED((np_dst, d), jnp.float32) for np_dst, _ in meta]
    )

    nj = len(jobs)

    def body(*refs):
        if deg_mode:
            dst1d, _tok, out0 = refs[0], refs[1], refs[2]
            sc = refs[3:]
            outs = (out0,)
            tabs = (None,)
            srcs = (None,)
            dsts = (dst1d,)
        else:
            tabs = refs[0:3 * nj:3]
            srcs = refs[1:3 * nj:3]
            dsts = refs[2:3 * nj:3]
            outs = refs[3 * nj + 1:4 * nj + 1]
            sc = refs[4 * nj + 1:]
        (idx_s, idx_d, rows, obuf) = sc[0:4]
        accs = sc[4:]
        c = lax.axis_index("c")
        s = lax.axis_index("s")
        wid = c * NS + s

        zeros16 = jnp.zeros((16,), jnp.float32)

        def zrow(i, carry):
            for j in range(d // 16):
                obuf[i, pl.ds(j * 16, 16)] = zeros16
            return carry
        lax.fori_loop(0, OCH, zrow, 0)

        if deg_mode:
            ones16 = jnp.ones((16,), jnp.float32)

            def orow(i, carry):
                for j in range(d // 16):
                    rows[i, pl.ds(j * 16, 16)] = ones16
                return carry
            lax.fori_loop(0, CHUNK, orow, 0)

        for (np_dst, _), acc in zip(meta, accs):
            rows_sub = np_dst // NS
            r0 = s * rows_sub
            nfull = rows_sub // OCH
            rem = rows_sub % OCH

            def zcopy(i, carry, acc=acc, r0=r0):
                pltpu.sync_copy(obuf, acc.at[pl.ds(r0 + i * OCH, OCH)])
                return carry
            lax.fori_loop(0, nfull, zcopy, 0)
            if rem:
                pltpu.sync_copy(obuf.at[pl.ds(0, rem)],
                                acc.at[pl.ds(r0 + nfull * OCH, rem)])
        plsc.subcore_barrier()
        plsc.subcore_barrier()

        for ji in range(nj):
            np_dst, per_w = meta[ji]
            table, src3d, dst3d, acc = tabs[ji], srcs[ji], dsts[ji], accs[ji]
            nchunk = per_w // CHUNK
            base = wid * per_w

            def chunk_body(k, carry, acc=acc, table=table,
                           src3d=src3d, dst3d=dst3d):
                b = base + k * CHUNK
                pltpu.sync_copy(dst3d.at[pl.ds(b, CHUNK)], idx_d)
                if not deg_mode:
                    pltpu.sync_copy(src3d.at[pl.ds(b, CHUNK)], idx_s)
                    pltpu.sync_copy(table.at[idx_s], rows)
                pltpu.sync_copy(rows, acc.at[idx_d], add=True)
                return carry
            lax.fori_loop(0, nchunk, chunk_body, 0)

        plsc.subcore_barrier()
        plsc.subcore_barrier()

        for (np_dst, _), acc, out in zip(meta, accs, outs):
            rows_sub = np_dst // NS
            r0 = s * rows_sub
            nfull = rows_sub // OCH
            rem = rows_sub % OCH

            def ocopy(i, carry, acc=acc, out=out, r0=r0):
                pltpu.sync_copy(acc.at[pl.ds(r0 + i * OCH, OCH)], obuf)
                pltpu.sync_copy(obuf,
                                out.at[c, pl.ds(r0 + i * OCH, OCH)])
                return carry
            lax.fori_loop(0, nfull, ocopy, 0)
            if rem:
                pltpu.sync_copy(acc.at[pl.ds(r0 + nfull * OCH, rem)],
                                obuf.at[pl.ds(0, rem)])
                pltpu.sync_copy(obuf.at[pl.ds(0, rem)],
                                out.at[c, pl.ds(r0 + nfull * OCH, rem)])

    out_type = tuple(jax.ShapeDtypeStruct((NC, np_dst, d), jnp.float32)
                     for np_dst, _ in meta)
    if len(out_type) == 1:
        out_type = out_type[0]
    fn = pl.kernel(body, out_type=out_type, mesh=mesh, scratch_types=scratch)
    return fn


def _pad_edges(src, dst, n_edges, dummy_row):
    _, e_pad = _job_pad(n_edges)
    pe = e_pad - src.shape[0]
    src = jnp.concatenate([src, jnp.zeros((pe,), jnp.int32)])
    dst = jnp.concatenate([dst, jnp.full((pe,), dummy_row, jnp.int32)])
    return src, dst


_BP = 1000


def _dinv_from(degp_ref):
    deg = degp_ref[0, :, 0:1] + degp_ref[1, :, 0:1] + 1.0
    return lax.rsqrt(deg)


def _pre_paper_body(x_ref, degp_ref, wcat_ref, wg_ref, his_ref, rr_ref, hs_ref):
    x = x_ref[...]
    dinv = _dinv_from(degp_ref)
    y = jnp.dot(x, wcat_ref[...], preferred_element_type=jnp.float32)
    his_ref[...] = y[:, :HIDDEN]
    rr_ref[...] = y[:, HIDDEN:]
    hs_ref[...] = jnp.dot(x * dinv, wg_ref[...],
                          preferred_element_type=jnp.float32)


def _pre_label_body(x_ref, wcat_ref, hrev_ref, ri_ref):
    y = jnp.dot(x_ref[...], wcat_ref[...], preferred_element_type=jnp.float32)
    hrev_ref[...] = y[:, :HIDDEN]
    ri_ref[...] = y[:, HIDDEN:]


def _mid_paper_body(sc_ref, sr_ref, hs_ref, rr_ref, degp_ref, bg_ref, brr_ref,
                    zp1_ref, zp1s_ref):
    dinv = _dinv_from(degp_ref)
    gcn = dinv * (sc_ref[0] + sc_ref[1] + hs_ref[...]) + bg_ref[...]
    rev = sr_ref[0] + sr_ref[1] + brr_ref[...] + rr_ref[...]
    zp1 = jax.nn.relu(0.5 * (gcn + rev))
    zp1_ref[...] = zp1
    zp1s_ref[...] = zp1 * dinv


def _mid_label_body(sis_ref, ri_ref, bri_ref, zl1_ref):
    zl1_ref[...] = jax.nn.relu(sis_ref[0] + sis_ref[1] + bri_ref[...]
                               + ri_ref[...])


def _post_paper_body(sc_ref, sr_ref, zp1s_ref, zp1_ref, degp_ref, bg_ref,
                     brr_ref, wg2_ref, wrr2_ref, wtr2_ref, zp_ref):
    dinv = _dinv_from(degp_ref)
    a = sc_ref[0] + sc_ref[1] + zp1s_ref[...]
    gcn = dinv * jnp.dot(a, wg2_ref[...], preferred_element_type=jnp.float32) \
        + bg_ref[...]
    rev = jnp.dot(sr_ref[0] + sr_ref[1], wrr2_ref[...],
                  preferred_element_type=jnp.float32) + brr_ref[...] \
        + jnp.dot(zp1_ref[...], wtr2_ref[...],
                  preferred_element_type=jnp.float32)
    zp_ref[...] = 0.5 * (gcn + rev)


def _post_label_body(sis_ref, zl1_ref, bri_ref, wri2_ref, wti2_ref, zl_ref):
    zl_ref[...] = jnp.dot(sis_ref[0] + sis_ref[1], wri2_ref[...],
                          preferred_element_type=jnp.float32) \
        + bri_ref[...] \
        + jnp.dot(zl1_ref[...], wti2_ref[...],
                  preferred_element_type=jnp.float32)


def _row_spec(d):
    return pl.BlockSpec((_BP, d), lambda i: (i, 0))


def _part_spec(d):
    return pl.BlockSpec((NC, _BP, d), lambda i: (0, i, 0))


def _full_spec(shape):
    nz = len(shape)
    return pl.BlockSpec(shape, lambda *a: (0,) * nz)


def kernel(x_paper, x_label, edge_index_cites, is_src, is_dst, rev_src, rev_dst,
           W_gcn1, b_gcn1, Wrel_is1, brel_is1, Wroot_is1, Wrel_rev1, brel_rev1,
           Wroot_rev1, W_gcn2, b_gcn2, Wrel_is2, brel_is2, Wroot_is2,
           Wrel_rev2, brel_rev2, Wroot_rev2):
    f32 = jnp.float32
    cit_src = edge_index_cites[0]
    cit_dst = edge_index_cites[1]

    deg_fn = _make_sc_multi([(N_PAPER, E_CITES)], deg_mode=True)
    cites_fn = _make_sc_multi([(N_PAPER, E_CITES)])
    rev_fn = _make_sc_multi([(N_PAPER, E_IS)])
    is_fn = _make_sc_multi([(N_LABEL, E_IS)])

    csrc_p, cdst_p = _pad_edges(cit_src, cit_dst, E_CITES, N_PAPER)
    rsrc_p, rdst_p = _pad_edges(rev_src, rev_dst, E_IS, N_PAPER)
    isrc_p, idst_p = _pad_edges(is_src, is_dst, E_IS, N_LABEL)

    tok0 = x_paper[:8, :16]
    degp_full = deg_fn(cdst_p, tok0)
    degp = degp_full[:, :N_PAPER, :]

    def _tok(arr):
        return arr[0, :8, :16]

    wcat_p1 = jnp.concatenate([Wrel_is1, Wroot_rev1], axis=1)
    grid_p = (N_PAPER // _BP,)
    his1, rr1, hs1 = pl.pallas_call(
        _pre_paper_body,
        grid=grid_p,
        in_specs=[_row_spec(D_FEAT),
                  pl.BlockSpec((NC, _BP, 16), lambda i: (0, i, 0)),
                  _full_spec((D_FEAT, 2 * HIDDEN)),
                  _full_spec((D_FEAT, HIDDEN))],
        out_specs=[_row_spec(HIDDEN)] * 3,
        out_shape=[jax.ShapeDtypeStruct((N_PAPER, HIDDEN), f32)] * 3,
    )(x_paper, degp, wcat_p1, W_gcn1)

    wcat_l1 = jnp.concatenate([Wrel_rev1, Wroot_is1], axis=1)
    hrev1, ri1 = pl.pallas_call(
        _pre_label_body,
        in_specs=[_full_spec((N_LABEL, D_FEAT)),
                  _full_spec((D_FEAT, 2 * HIDDEN))],
        out_specs=[_full_spec((N_LABEL, HIDDEN))] * 2,
        out_shape=[jax.ShapeDtypeStruct((N_LABEL, HIDDEN), f32)] * 2,
    )(x_label, wcat_l1)

    sc1f = cites_fn(hs1, csrc_p, cdst_p, _tok(degp_full))
    sr1f = rev_fn(hrev1, rsrc_p, rdst_p, _tok(sc1f))
    sis1f = is_fn(his1, isrc_p, idst_p, _tok(sr1f))
    sc1 = sc1f[:, :N_PAPER, :]
    sr1 = sr1f[:, :N_PAPER, :]
    sis1 = sis1f[:, :N_LABEL, :]

    zp1, zp1s = pl.pallas_call(
        _mid_paper_body,
        grid=grid_p,
        in_specs=[_part_spec(HIDDEN), _part_spec(HIDDEN),
                  _row_spec(HIDDEN), _row_spec(HIDDEN),
                  pl.BlockSpec((NC, _BP, 16), lambda i: (0, i, 0)),
                  _full_spec((1, HIDDEN)), _full_spec((1, HIDDEN))],
        out_specs=[_row_spec(HIDDEN)] * 2,
        out_shape=[jax.ShapeDtypeStruct((N_PAPER, HIDDEN), f32)] * 2,
    )(sc1, sr1, hs1, rr1, degp, b_gcn1.reshape(1, -1),
      brel_rev1.reshape(1, -1))

    zl1 = pl.pallas_call(
        _mid_label_body,
        in_specs=[_full_spec((NC, N_LABEL, HIDDEN)),
                  _full_spec((N_LABEL, HIDDEN)),
                  _full_spec((1, HIDDEN))],
        out_specs=_full_spec((N_LABEL, HIDDEN)),
        out_shape=jax.ShapeDtypeStruct((N_LABEL, HIDDEN), f32),
    )(sis1, ri1, brel_is1.reshape(1, -1))

    sc2f = cites_fn(zp1s, csrc_p, cdst_p, _tok(sis1f))
    sr2f = rev_fn(zl1, rsrc_p, rdst_p, _tok(sc2f))
    sis2f = is_fn(zp1, isrc_p, idst_p, _tok(sr2f))
    sc2 = sc2f[:, :N_PAPER, :]
    sr2 = sr2f[:, :N_PAPER, :]
    sis2 = sis2f[:, :N_LABEL, :]

    zp2 = pl.pallas_call(
        _post_paper_body,
        grid=grid_p,
        in_specs=[_part_spec(HIDDEN), _part_spec(HIDDEN),
                  _row_spec(HIDDEN), _row_spec(HIDDEN),
                  pl.BlockSpec((NC, _BP, 16), lambda i: (0, i, 0)),
                  _full_spec((1, EMB)), _full_spec((1, EMB)),
                  _full_spec((HIDDEN, EMB)), _full_spec((HIDDEN, EMB)),
                  _full_spec((HIDDEN, EMB))],
        out_specs=_row_spec(EMB),
        out_shape=jax.ShapeDtypeStruct((N_PAPER, EMB), f32),
    )(sc2, sr2, zp1s, zp1, degp, b_gcn2.reshape(1, -1),
      brel_rev2.reshape(1, -1), W_gcn2, Wrel_rev2, Wroot_rev2)

    zl2 = pl.pallas_call(
        _post_label_body,
        in_specs=[_full_spec((NC, N_LABEL, HIDDEN)),
                  _full_spec((N_LABEL, HIDDEN)),
                  _full_spec((1, EMB)),
                  _full_spec((HIDDEN, EMB)), _full_spec((HIDDEN, EMB))],
        out_specs=_full_spec((N_LABEL, EMB)),
        out_shape=jax.ShapeDtypeStruct((N_LABEL, EMB), f32),
    )(sis2, zl1, brel_is2.reshape(1, -1), Wrel_is2, Wroot_is2)

    return zp2, zl2
